# baseline (speedup 1.0000x reference)
import functools

import jax
import jax.numpy as jnp
from jax import lax
from jax.experimental import pallas as pl
from jax.experimental.pallas import tpu as pltpu
from jax.experimental.pallas import tpu_sc as plsc

N = 10000
E = 320000
D = 128
UNIT = 128
R = E // UNIT
NSC = 2
NSUB = 16
NW = NSC * NSUB
NPB = 400
EPB = 1600
K = 2
EC = E // K
RC = EC // UNIT

_relu = jax.nn.relu


def _pack(a):
    b = a.astype(jnp.bfloat16)
    u = lax.convert_element_type(
        lax.bitcast_convert_type(b, jnp.uint16), jnp.uint32)
    w = u[:, :32] | (u[:, 32:] << 16)
    return lax.bitcast_convert_type(w, jnp.float32)


def _unpack(p):
    u = lax.bitcast_convert_type(p, jnp.uint32)
    lo = lax.bitcast_convert_type(u << 16, jnp.float32)
    hi = lax.bitcast_convert_type(u & jnp.uint32(0xFFFF0000), jnp.float32)
    return jnp.concatenate([lo, hi], axis=1)


def _prep_body(x_ref, posp_ref, WinT, b_in, WlinT, WsrcT, WdstT, P1pT, pb1,
               P2T, pb2, A1T, ab1, A2T, ab2,
               T1_ref, T2_ref, den0_ref, num0_ref):
    x = x_ref[...]
    h = _relu(jnp.dot(x, WinT[...], preferred_element_type=jnp.float32)
              + b_in[...])
    xl = jnp.dot(h, WlinT[...], preferred_element_type=jnp.float32)
    dd = jnp.dot(jnp.dot(h, WdstT[...], preferred_element_type=jnp.float32),
                 A1T[...], preferred_element_type=jnp.float32)
    ss = jnp.dot(jnp.dot(h, WsrcT[...], preferred_element_type=jnp.float32),
                 A1T[...], preferred_element_type=jnp.float32)
    q = jnp.dot(posp_ref[...], P1pT[...], preferred_element_type=jnp.float32)
    dl64 = _relu(pb1[...])
    dl128 = _relu(jnp.dot(dl64, P2T[...],
                          preferred_element_type=jnp.float32) + pb2[...])
    v0 = _relu(jnp.dot(dl128, A1T[...], preferred_element_type=jnp.float32)
               + dd - ss + ab1[...])
    alpha0 = _relu(jnp.dot(v0, A2T[...], preferred_element_type=jnp.float32)
                   + ab2[...])
    ex0 = jnp.exp(alpha0)
    den0_ref[...] = ex0
    num0_ref[...] = ex0 * (xl + dl128)
    T1_ref[...] = jnp.concatenate([_pack(q), _pack(dd)], axis=1)
    T2_ref[...] = jnp.concatenate([xl, _pack(q), _pack(ss)], axis=1)


def _prep_call(x, posp, WinT, b_in, WlinT, WsrcT, WdstT, P1pT, pb1, P2T, pb2,
               A1T, ab1, A2T, ab2):
    nb = N // NPB
    full = lambda a: pl.BlockSpec(a.shape, lambda i: (0,) * a.ndim)
    row_spec = lambda w: pl.BlockSpec((NPB, w), lambda i: (i, 0))
    return pl.pallas_call(
        _prep_body,
        grid=(nb,),
        in_specs=[row_spec(D), row_spec(8)] + [
            full(a) for a in (WinT, b_in, WlinT, WsrcT, WdstT, P1pT, pb1,
                              P2T, pb2, A1T, ab1, A2T, ab2)],
        out_specs=[row_spec(64), row_spec(192), row_spec(D), row_spec(D)],
        out_shape=[jax.ShapeDtypeStruct((N, 64), jnp.float32),
                   jax.ShapeDtypeStruct((N, 192), jnp.float32),
                   jax.ShapeDtypeStruct((N, D), jnp.float32),
                   jax.ShapeDtypeStruct((N, D), jnp.float32)],
    )(x, posp, WinT, b_in, WlinT, WsrcT, WdstT, P1pT, pb1, P2T, pb2,
      A1T, ab1, A2T, ab2)


_sc_mesh = plsc.VectorSubcoreMesh(core_axis_name="c", subcore_axis_name="s")


@functools.partial(
    pl.kernel,
    mesh=_sc_mesh,
    out_type=[jax.ShapeDtypeStruct((EC, 64), jnp.float32),
              jax.ShapeDtypeStruct((EC, 192), jnp.float32)],
    scratch_types=[pltpu.VMEM((UNIT,), jnp.int32),
                   pltpu.VMEM((UNIT,), jnp.int32),
                   pltpu.VMEM((UNIT, 64), jnp.float32),
                   pltpu.VMEM((UNIT, 192), jnp.float32),
                   pltpu.SemaphoreType.DMA],
    compiler_params=pltpu.CompilerParams(use_tc_tiling_on_sc=False),
)
def _sc_gather(src1, dst1, T1, T2, outD, outS, idx_s, idx_d, bufD, bufS,
               sem):
    c = lax.axis_index("c")
    s = lax.axis_index("s")
    wid = s * NSC + c
    base = RC // NW
    nrows = base + jnp.where(wid < RC - base * NW, 1, 0)

    def body(i, carry):
        row = wid + i * NW
        off = pl.multiple_of(row * UNIT, UNIT)
        pltpu.sync_copy(src1.at[pl.ds(off, UNIT)], idx_s)
        pltpu.sync_copy(dst1.at[pl.ds(off, UNIT)], idx_d)
        cp1 = pltpu.async_copy(T1.at[idx_d], bufD, sem)
        cp2 = pltpu.async_copy(T2.at[idx_s], bufS, sem)
        cp1.wait()
        cp2.wait()
        pltpu.sync_copy(bufD, outD.at[pl.ds(off, UNIT), :])
        pltpu.sync_copy(bufS, outS.at[pl.ds(off, UNIT), :])
        return carry

    lax.fori_loop(0, nrows, body, 0)


def _emlp_body(eD_ref, eS_ref, pb1, P2T, pb2, A1T, ab1, A2T, ab2, out_ref):
    eD = eD_ref[...]
    eS = eS_ref[...]
    qd = _unpack(eD[:, :32])
    dd = _unpack(eD[:, 32:])
    xl = eS[:, :128]
    qs = _unpack(eS[:, 128:160])
    ss = _unpack(eS[:, 160:192])
    e1 = _relu(qd - qs + pb1[...])
    delta = _relu(jnp.dot(e1, P2T[...], preferred_element_type=jnp.float32)
                  + pb2[...])
    v = _relu(jnp.dot(delta, A1T[...], preferred_element_type=jnp.float32)
              + (dd - ss) + ab1[...])
    alpha = _relu(jnp.dot(v, A2T[...], preferred_element_type=jnp.float32)
                  + ab2[...])
    ex = jnp.exp(alpha)
    msg = ex * (xl + delta)
    out_ref[0] = jnp.concatenate([ex[:, :64], msg[:, :64]], axis=1)
    out_ref[1] = jnp.concatenate([ex[:, 64:], msg[:, 64:]], axis=1)


def _emlp_call(eD, eS, pb1, P2T, pb2, A1T, ab1, A2T, ab2):
    nb = EC // EPB
    full = lambda a: pl.BlockSpec(a.shape, lambda i: (0,) * a.ndim)
    return pl.pallas_call(
        _emlp_body,
        grid=(nb,),
        in_specs=[pl.BlockSpec((EPB, 64), lambda i: (i, 0)),
                  pl.BlockSpec((EPB, 192), lambda i: (i, 0))] + [
            full(a) for a in (pb1, P2T, pb2, A1T, ab1, A2T, ab2)],
        out_specs=pl.BlockSpec((2, EPB, D), lambda i: (0, i, 0)),
        out_shape=jax.ShapeDtypeStruct((2, EC, D), jnp.float32),
    )(eD, eS, pb1, P2T, pb2, A1T, ab1, A2T, ab2)


_RSLICE = 624


@functools.partial(
    pl.kernel,
    mesh=_sc_mesh,
    out_type=jax.ShapeDtypeStruct((2, N, D), jnp.float32),
    scratch_types=[pltpu.VMEM((UNIT,), jnp.int32),
                   pltpu.VMEM((UNIT, D), jnp.float32),
                   pltpu.VMEM_SHARED((N, D), jnp.float32)],
)
def _sc_scatter(dst1, em3, zeros, accO, idx_d, bufE, acc_sh):
    c = lax.axis_index("c")
    s = lax.axis_index("s")
    r0 = s * _RSLICE
    pltpu.sync_copy(zeros, acc_sh.at[pl.ds(r0, _RSLICE), :])

    @pl.when(s == NSUB - 1)
    def _():
        pltpu.sync_copy(zeros.at[pl.ds(0, 16), :],
                        acc_sh.at[pl.ds(NSUB * _RSLICE, 16), :])

    plsc.subcore_barrier()
    sbase = RC // NSUB
    nrows = sbase + jnp.where(s < RC - sbase * NSUB, 1, 0)

    def body(i, carry):
        row = s + i * NSUB
        off = pl.multiple_of(row * UNIT, UNIT)
        pltpu.sync_copy(dst1.at[pl.ds(off, UNIT)], idx_d)
        pltpu.sync_copy(em3.at[c, pl.ds(off, UNIT), :], bufE)
        pltpu.sync_copy(bufE, acc_sh.at[idx_d], add=True)
        return carry

    lax.fori_loop(0, nrows, body, 0)
    plsc.subcore_barrier()
    pltpu.sync_copy(acc_sh.at[pl.ds(r0, _RSLICE), :],
                    accO.at[c, pl.ds(r0, _RSLICE), :])

    @pl.when(s == NSUB - 1)
    def _():
        pltpu.sync_copy(acc_sh.at[pl.ds(NSUB * _RSLICE, 16), :],
                        accO.at[c, pl.ds(NSUB * _RSLICE, 16), :])


def _final_body(*refs):
    acc_refs = refs[:K]
    den0_ref, num0_ref, WoutT, b_out, o_ref = refs[K:]
    acc = acc_refs[0][...]
    for r in acc_refs[1:]:
        acc = acc + r[...]
    den = jnp.concatenate([acc[0, :, :64], acc[1, :, :64]], axis=1)
    num = jnp.concatenate([acc[0, :, 64:], acc[1, :, 64:]], axis=1)
    den = den + den0_ref[...] + 1e-16
    num = num + num0_ref[...]
    o_ref[...] = _relu(jnp.dot(num / den, WoutT[...],
                               preferred_element_type=jnp.float32)
                       + b_out[...])


def _final_call(accs, den0, num0, WoutT, b_out):
    nb = N // NPB
    full = lambda a: pl.BlockSpec(a.shape, lambda i: (0,) * a.ndim)
    row_spec = pl.BlockSpec((NPB, D), lambda i: (i, 0))
    acc_spec = pl.BlockSpec((2, NPB, D), lambda i: (0, i, 0))
    return pl.pallas_call(
        _final_body,
        grid=(nb,),
        in_specs=[acc_spec] * K + [row_spec, row_spec, full(WoutT),
                                   full(b_out)],
        out_specs=row_spec,
        out_shape=jax.ShapeDtypeStruct((N, D), jnp.float32),
    )(*accs, den0, num0, WoutT, b_out)


def kernel(x, pos, edge_index, Win, b_in, Wout, b_out, Wlin, Wsrc, Wdst,
           P1, pb1, P2, pb2, A1, ab1, A2, ab2):
    posp = jnp.pad(pos, ((0, 0), (0, 5)))
    P1pT = jnp.pad(P1, ((0, 0), (0, 5))).T
    row = lambda v: v.reshape(1, -1)
    T1, T2, den0, num0 = _prep_call(
        x, posp, Win.T, row(b_in), Wlin.T, Wsrc.T, Wdst.T, P1pT, row(pb1),
        P2.T, row(pb2), A1.T, row(ab1), A2.T, row(ab2))
    src1 = edge_index[0]
    dst1 = edge_index[1]
    zeros = jnp.zeros((_RSLICE, D), jnp.float32)
    accs = []
    for k in range(K):
        sl = slice(k * EC, (k + 1) * EC)
        eD, eS = _sc_gather(src1[sl], dst1[sl], T1, T2)
        em3 = _emlp_call(eD, eS, row(pb1), P2.T, row(pb2), A1.T, row(ab1),
                         A2.T, row(ab2))
        accs.append(_sc_scatter(dst1[sl], em3, zeros))
    return _final_call(accs, den0, num0, Wout.T, row(b_out))


# speedup vs baseline: 1.4270x; 1.4270x over previous
import functools

import jax
import jax.numpy as jnp
from jax import lax
from jax.experimental import pallas as pl
from jax.experimental.pallas import tpu as pltpu
from jax.experimental.pallas import tpu_sc as plsc

N = 10000
E = 320000
D = 128
UNIT = 128
R = E // UNIT
NSC = 2
NSUB = 16
NW = NSC * NSUB
NPB = 400
EPB = 1600
K = 2
EC = E // K
RC = EC // UNIT

_relu = jax.nn.relu


def _pack(a):
    n = a.shape[1] // 2
    b = a.astype(jnp.bfloat16)
    u = lax.convert_element_type(
        lax.bitcast_convert_type(b, jnp.uint16), jnp.uint32)
    w = u[:, :n] | (u[:, n:] << 16)
    return lax.bitcast_convert_type(w, jnp.float32)


def _unpack(p):
    u = lax.bitcast_convert_type(p, jnp.uint32)
    lo = lax.bitcast_convert_type(u << 16, jnp.float32)
    hi = lax.bitcast_convert_type(u & jnp.uint32(0xFFFF0000), jnp.float32)
    return jnp.concatenate([lo, hi], axis=1)


def _prep_body(x_ref, posp_ref, WinT, b_in, WlinT, WsrcT, WdstT, P1pT, pb1,
               P2T, pb2, A1T, ab1, A2T, ab2,
               T1_ref, T2_ref, den0_ref, num0_ref):
    x = x_ref[...]
    h = _relu(jnp.dot(x, WinT[...], preferred_element_type=jnp.float32)
              + b_in[...])
    xl = jnp.dot(h, WlinT[...], preferred_element_type=jnp.float32)
    dd = jnp.dot(jnp.dot(h, WdstT[...], preferred_element_type=jnp.float32),
                 A1T[...], preferred_element_type=jnp.float32)
    ss = jnp.dot(jnp.dot(h, WsrcT[...], preferred_element_type=jnp.float32),
                 A1T[...], preferred_element_type=jnp.float32)
    q = jnp.dot(posp_ref[...], P1pT[...], preferred_element_type=jnp.float32)
    dl64 = _relu(pb1[...])
    dl128 = _relu(jnp.dot(dl64, P2T[...],
                          preferred_element_type=jnp.float32) + pb2[...])
    v0 = _relu(jnp.dot(dl128, A1T[...], preferred_element_type=jnp.float32)
               + dd - ss + ab1[...])
    alpha0 = _relu(jnp.dot(v0, A2T[...], preferred_element_type=jnp.float32)
                   + ab2[...])
    ex0 = jnp.exp(alpha0)
    den0_ref[...] = ex0
    num0_ref[...] = ex0 * (xl + dl128)
    T1_ref[...] = jnp.concatenate([q, dd], axis=1)
    T2_ref[...] = jnp.concatenate([_pack(xl), _pack(q), _pack(ss)], axis=1)


def _prep_call(x, posp, WinT, b_in, WlinT, WsrcT, WdstT, P1pT, pb1, P2T, pb2,
               A1T, ab1, A2T, ab2):
    nb = N // NPB
    full = lambda a: pl.BlockSpec(a.shape, lambda i: (0,) * a.ndim)
    row_spec = lambda w: pl.BlockSpec((NPB, w), lambda i: (i, 0))
    return pl.pallas_call(
        _prep_body,
        grid=(nb,),
        in_specs=[row_spec(D), row_spec(8)] + [
            full(a) for a in (WinT, b_in, WlinT, WsrcT, WdstT, P1pT, pb1,
                              P2T, pb2, A1T, ab1, A2T, ab2)],
        out_specs=[row_spec(D), row_spec(D), row_spec(D), row_spec(D)],
        out_shape=[jax.ShapeDtypeStruct((N, D), jnp.float32),
                   jax.ShapeDtypeStruct((N, D), jnp.float32),
                   jax.ShapeDtypeStruct((N, D), jnp.float32),
                   jax.ShapeDtypeStruct((N, D), jnp.float32)],
    )(x, posp, WinT, b_in, WlinT, WsrcT, WdstT, P1pT, pb1, P2T, pb2,
      A1T, ab1, A2T, ab2)


_sc_mesh = plsc.VectorSubcoreMesh(core_axis_name="c", subcore_axis_name="s")


@functools.partial(
    pl.kernel,
    mesh=_sc_mesh,
    out_type=[jax.ShapeDtypeStruct((EC, D), jnp.float32),
              jax.ShapeDtypeStruct((EC, D), jnp.float32)],
    scratch_types=[pltpu.VMEM((UNIT,), jnp.int32),
                   pltpu.VMEM((UNIT,), jnp.int32),
                   pltpu.VMEM((UNIT, D), jnp.float32),
                   pltpu.VMEM((UNIT, D), jnp.float32),
                   pltpu.SemaphoreType.DMA],
)
def _sc_gather(src1, dst1, T1, T2, outD, outS, idx_s, idx_d, bufD, bufS,
               sem):
    c = lax.axis_index("c")
    s = lax.axis_index("s")
    wid = s * NSC + c
    base = RC // NW
    nrows = base + jnp.where(wid < RC - base * NW, 1, 0)

    def body(i, carry):
        row = wid + i * NW
        off = pl.multiple_of(row * UNIT, UNIT)
        pltpu.sync_copy(src1.at[pl.ds(off, UNIT)], idx_s)
        pltpu.sync_copy(dst1.at[pl.ds(off, UNIT)], idx_d)
        cp1 = pltpu.async_copy(T1.at[idx_d], bufD, sem)
        cp2 = pltpu.async_copy(T2.at[idx_s], bufS, sem)
        cp1.wait()
        cp2.wait()
        pltpu.sync_copy(bufD, outD.at[pl.ds(off, UNIT), :])
        pltpu.sync_copy(bufS, outS.at[pl.ds(off, UNIT), :])
        return carry

    lax.fori_loop(0, nrows, body, 0)


def _emlp_body(eD_ref, eS_ref, pb1, P2T, pb2, A1T, ab1, A2T, ab2, out_ref):
    eD = eD_ref[...]
    eS = eS_ref[...]
    qd = eD[:, :64]
    dd = eD[:, 64:]
    xl = _unpack(eS[:, :64])
    qs = _unpack(eS[:, 64:96])
    ss = _unpack(eS[:, 96:128])
    e1 = _relu(qd - qs + pb1[...])
    delta = _relu(jnp.dot(e1, P2T[...], preferred_element_type=jnp.float32)
                  + pb2[...])
    v = _relu(jnp.dot(delta, A1T[...], preferred_element_type=jnp.float32)
              + (dd - ss) + ab1[...])
    alpha = _relu(jnp.dot(v, A2T[...], preferred_element_type=jnp.float32)
                  + ab2[...])
    ex = jnp.exp(alpha)
    msg = ex * (xl + delta)
    out_ref[0] = jnp.concatenate([ex[:, :64], msg[:, :64]], axis=1)
    out_ref[1] = jnp.concatenate([ex[:, 64:], msg[:, 64:]], axis=1)


def _emlp_call(eD, eS, pb1, P2T, pb2, A1T, ab1, A2T, ab2):
    nb = EC // EPB
    full = lambda a: pl.BlockSpec(a.shape, lambda i: (0,) * a.ndim)
    return pl.pallas_call(
        _emlp_body,
        grid=(nb,),
        in_specs=[pl.BlockSpec((EPB, D), lambda i: (i, 0)),
                  pl.BlockSpec((EPB, D), lambda i: (i, 0))] + [
            full(a) for a in (pb1, P2T, pb2, A1T, ab1, A2T, ab2)],
        out_specs=pl.BlockSpec((2, EPB, D), lambda i: (0, i, 0)),
        out_shape=jax.ShapeDtypeStruct((2, EC, D), jnp.float32),
    )(eD, eS, pb1, P2T, pb2, A1T, ab1, A2T, ab2)


_RSLICE = 624


@functools.partial(
    pl.kernel,
    mesh=_sc_mesh,
    out_type=jax.ShapeDtypeStruct((2, N, D), jnp.float32),
    scratch_types=[pltpu.VMEM((UNIT,), jnp.int32),
                   pltpu.VMEM((UNIT, D), jnp.float32),
                   pltpu.VMEM_SHARED((N, D), jnp.float32)],
)
def _sc_scatter(dst1, em3, zeros, accO, idx_d, bufE, acc_sh):
    c = lax.axis_index("c")
    s = lax.axis_index("s")
    r0 = s * _RSLICE
    pltpu.sync_copy(zeros, acc_sh.at[pl.ds(r0, _RSLICE), :])

    @pl.when(s == NSUB - 1)
    def _():
        pltpu.sync_copy(zeros.at[pl.ds(0, 16), :],
                        acc_sh.at[pl.ds(NSUB * _RSLICE, 16), :])

    plsc.subcore_barrier()
    sbase = RC // NSUB
    nrows = sbase + jnp.where(s < RC - sbase * NSUB, 1, 0)

    def body(i, carry):
        row = s + i * NSUB
        off = pl.multiple_of(row * UNIT, UNIT)
        pltpu.sync_copy(dst1.at[pl.ds(off, UNIT)], idx_d)
        pltpu.sync_copy(em3.at[c, pl.ds(off, UNIT), :], bufE)
        pltpu.sync_copy(bufE, acc_sh.at[idx_d], add=True)
        return carry

    lax.fori_loop(0, nrows, body, 0)
    plsc.subcore_barrier()
    pltpu.sync_copy(acc_sh.at[pl.ds(r0, _RSLICE), :],
                    accO.at[c, pl.ds(r0, _RSLICE), :])

    @pl.when(s == NSUB - 1)
    def _():
        pltpu.sync_copy(acc_sh.at[pl.ds(NSUB * _RSLICE, 16), :],
                        accO.at[c, pl.ds(NSUB * _RSLICE, 16), :])


def _final_body(*refs):
    acc_refs = refs[:K]
    den0_ref, num0_ref, WoutT, b_out, o_ref = refs[K:]
    acc = acc_refs[0][...]
    for r in acc_refs[1:]:
        acc = acc + r[...]
    den = jnp.concatenate([acc[0, :, :64], acc[1, :, :64]], axis=1)
    num = jnp.concatenate([acc[0, :, 64:], acc[1, :, 64:]], axis=1)
    den = den + den0_ref[...] + 1e-16
    num = num + num0_ref[...]
    o_ref[...] = _relu(jnp.dot(num / den, WoutT[...],
                               preferred_element_type=jnp.float32)
                       + b_out[...])


def _final_call(accs, den0, num0, WoutT, b_out):
    nb = N // NPB
    full = lambda a: pl.BlockSpec(a.shape, lambda i: (0,) * a.ndim)
    row_spec = pl.BlockSpec((NPB, D), lambda i: (i, 0))
    acc_spec = pl.BlockSpec((2, NPB, D), lambda i: (0, i, 0))
    return pl.pallas_call(
        _final_body,
        grid=(nb,),
        in_specs=[acc_spec] * K + [row_spec, row_spec, full(WoutT),
                                   full(b_out)],
        out_specs=row_spec,
        out_shape=jax.ShapeDtypeStruct((N, D), jnp.float32),
    )(*accs, den0, num0, WoutT, b_out)


def kernel(x, pos, edge_index, Win, b_in, Wout, b_out, Wlin, Wsrc, Wdst,
           P1, pb1, P2, pb2, A1, ab1, A2, ab2):
    posp = jnp.pad(pos, ((0, 0), (0, 5)))
    P1pT = jnp.pad(P1, ((0, 0), (0, 5))).T
    row = lambda v: v.reshape(1, -1)
    T1, T2, den0, num0 = _prep_call(
        x, posp, Win.T, row(b_in), Wlin.T, Wsrc.T, Wdst.T, P1pT, row(pb1),
        P2.T, row(pb2), A1.T, row(ab1), A2.T, row(ab2))
    src1 = edge_index[0]
    dst1 = edge_index[1]
    zeros = jnp.zeros((_RSLICE, D), jnp.float32)
    accs = []
    for k in range(K):
        sl = slice(k * EC, (k + 1) * EC)
        eD, eS = _sc_gather(src1[sl], dst1[sl], T1, T2)
        em3 = _emlp_call(eD, eS, row(pb1), P2.T, row(pb2), A1.T, row(ab1),
                         A2.T, row(ab2))
        accs.append(_sc_scatter(dst1[sl], em3, zeros))
    return _final_call(accs, den0, num0, Wout.T, row(b_out))


# speedup vs baseline: 1.7467x; 1.2240x over previous
import functools

import jax
import jax.numpy as jnp
from jax import lax
from jax.experimental import pallas as pl
from jax.experimental.pallas import tpu as pltpu
from jax.experimental.pallas import tpu_sc as plsc

N = 10000
E = 320000
D = 128
UNIT = 128
R = E // UNIT
NSC = 2
NSUB = 16
NW = NSC * NSUB
NPB = 400
EPB = 1600
K = 2
EC = E // K
RC = EC // UNIT

_relu = jax.nn.relu


def _pack(a):
    n = a.shape[1] // 2
    b = a.astype(jnp.bfloat16)
    u = lax.convert_element_type(
        lax.bitcast_convert_type(b, jnp.uint16), jnp.uint32)
    w = u[:, :n] | (u[:, n:] << 16)
    return lax.bitcast_convert_type(w, jnp.float32)


def _unpack(p):
    u = lax.bitcast_convert_type(p, jnp.uint32)
    lo = lax.bitcast_convert_type(u << 16, jnp.float32)
    hi = lax.bitcast_convert_type(u & jnp.uint32(0xFFFF0000), jnp.float32)
    return jnp.concatenate([lo, hi], axis=1)


def _prep_body(x_ref, posp_ref, WinT, b_in, WlinT, WsrcT, WdstT, P1pT, pb1,
               P2T, pb2, A1T, ab1, A2T, ab2,
               T1_ref, T2_ref, den0_ref, num0_ref):
    x = x_ref[...]
    h = _relu(jnp.dot(x, WinT[...], preferred_element_type=jnp.float32)
              + b_in[...])
    xl = jnp.dot(h, WlinT[...], preferred_element_type=jnp.float32)
    dd = jnp.dot(jnp.dot(h, WdstT[...], preferred_element_type=jnp.float32),
                 A1T[...], preferred_element_type=jnp.float32)
    ss = jnp.dot(jnp.dot(h, WsrcT[...], preferred_element_type=jnp.float32),
                 A1T[...], preferred_element_type=jnp.float32)
    q = jnp.dot(posp_ref[...], P1pT[...], preferred_element_type=jnp.float32)
    dl64 = _relu(pb1[...])
    dl128 = _relu(jnp.dot(dl64, P2T[...],
                          preferred_element_type=jnp.float32) + pb2[...])
    v0 = _relu(jnp.dot(dl128, A1T[...], preferred_element_type=jnp.float32)
               + dd - ss + ab1[...])
    alpha0 = _relu(jnp.dot(v0, A2T[...], preferred_element_type=jnp.float32)
                   + ab2[...])
    ex0 = jnp.exp(alpha0)
    den0_ref[...] = ex0
    num0_ref[...] = ex0 * (xl + dl128)
    T1_ref[...] = jnp.concatenate([q, dd], axis=1)
    T2_ref[...] = jnp.concatenate([_pack(xl), _pack(q), _pack(ss)], axis=1)


def _prep_call(x, posp, WinT, b_in, WlinT, WsrcT, WdstT, P1pT, pb1, P2T, pb2,
               A1T, ab1, A2T, ab2):
    nb = N // NPB
    full = lambda a: pl.BlockSpec(a.shape, lambda i: (0,) * a.ndim)
    row_spec = lambda w: pl.BlockSpec((NPB, w), lambda i: (i, 0))
    return pl.pallas_call(
        _prep_body,
        grid=(nb,),
        in_specs=[row_spec(D), row_spec(8)] + [
            full(a) for a in (WinT, b_in, WlinT, WsrcT, WdstT, P1pT, pb1,
                              P2T, pb2, A1T, ab1, A2T, ab2)],
        out_specs=[row_spec(D), row_spec(D), row_spec(D), row_spec(D)],
        out_shape=[jax.ShapeDtypeStruct((N, D), jnp.float32),
                   jax.ShapeDtypeStruct((N, D), jnp.float32),
                   jax.ShapeDtypeStruct((N, D), jnp.float32),
                   jax.ShapeDtypeStruct((N, D), jnp.float32)],
    )(x, posp, WinT, b_in, WlinT, WsrcT, WdstT, P1pT, pb1, P2T, pb2,
      A1T, ab1, A2T, ab2)


_sc_mesh = plsc.VectorSubcoreMesh(core_axis_name="c", subcore_axis_name="s")


_EPW = EC // NW
_GU = _EPW // UNIT
_GT = _EPW - _GU * UNIT


@functools.partial(
    pl.kernel,
    mesh=_sc_mesh,
    out_type=[jax.ShapeDtypeStruct((EC, D), jnp.float32),
              jax.ShapeDtypeStruct((EC, D), jnp.float32)],
    scratch_types=[pltpu.VMEM((UNIT,), jnp.int32),
                   pltpu.VMEM((UNIT,), jnp.int32),
                   pltpu.VMEM((UNIT,), jnp.int32),
                   pltpu.VMEM((UNIT,), jnp.int32),
                   pltpu.VMEM((UNIT, D), jnp.float32),
                   pltpu.VMEM((UNIT, D), jnp.float32),
                   pltpu.VMEM((UNIT, D), jnp.float32),
                   pltpu.VMEM((UNIT, D), jnp.float32),
                   pltpu.SemaphoreType.DMA,
                   pltpu.SemaphoreType.DMA],
)
def _sc_gather(src1, dst1, T1, T2, outD, outS, is0, id0, is1, id1,
               bD0, bS0, bD1, bS1, sem0, sem1):
    c = lax.axis_index("c")
    s = lax.axis_index("s")
    wid = s * NSC + c
    base = pl.multiple_of(wid * _EPW, 8)

    def start(off, isb, idb, bD, bS, sem):
        pltpu.sync_copy(src1.at[pl.ds(off, UNIT)], isb)
        pltpu.sync_copy(dst1.at[pl.ds(off, UNIT)], idb)
        pltpu.async_copy(T1.at[idb], bD, sem)
        pltpu.async_copy(T2.at[isb], bS, sem)

    def finish(off, isb, idb, bD, bS, sem):
        pltpu.make_async_copy(T1.at[idb], bD, sem).wait()
        pltpu.make_async_copy(T2.at[isb], bS, sem).wait()
        pltpu.sync_copy(bD, outD.at[pl.ds(off, UNIT), :])
        pltpu.sync_copy(bS, outS.at[pl.ds(off, UNIT), :])

    off_u = lambda u: pl.multiple_of(base + u * UNIT, 8)
    start(off_u(0), is0, id0, bD0, bS0, sem0)

    def body(t, carry):
        u = 2 * t
        start(off_u(u + 1), is1, id1, bD1, bS1, sem1)
        finish(off_u(u), is0, id0, bD0, bS0, sem0)
        start(off_u(u + 2), is0, id0, bD0, bS0, sem0)
        finish(off_u(u + 1), is1, id1, bD1, bS1, sem1)
        return carry

    lax.fori_loop(0, _GU // 2, body, 0)
    finish(off_u(_GU - 1), is0, id0, bD0, bS0, sem0)
    toff = pl.multiple_of(base + _GU * UNIT, 8)
    pltpu.sync_copy(src1.at[pl.ds(toff, _GT)], is1.at[pl.ds(0, _GT)])
    pltpu.sync_copy(dst1.at[pl.ds(toff, _GT)], id1.at[pl.ds(0, _GT)])
    cp1 = pltpu.async_copy(T1.at[id1.at[pl.ds(0, _GT)]],
                           bD1.at[pl.ds(0, _GT), :], sem1)
    cp2 = pltpu.async_copy(T2.at[is1.at[pl.ds(0, _GT)]],
                           bS1.at[pl.ds(0, _GT), :], sem1)
    cp1.wait()
    cp2.wait()
    pltpu.sync_copy(bD1.at[pl.ds(0, _GT), :], outD.at[pl.ds(toff, _GT), :])
    pltpu.sync_copy(bS1.at[pl.ds(0, _GT), :], outS.at[pl.ds(toff, _GT), :])


def _emlp_body(eD_ref, eS_ref, pb1, P2T, pb2, A1T, ab1, A2T, ab2, out_ref):
    eD = eD_ref[...]
    eS = eS_ref[...]
    qd = eD[:, :64]
    dd = eD[:, 64:]
    xl = _unpack(eS[:, :64])
    qs = _unpack(eS[:, 64:96])
    ss = _unpack(eS[:, 96:128])
    e1 = _relu(qd - qs + pb1[...])
    delta = _relu(jnp.dot(e1, P2T[...], preferred_element_type=jnp.float32)
                  + pb2[...])
    v = _relu(jnp.dot(delta, A1T[...], preferred_element_type=jnp.float32)
              + (dd - ss) + ab1[...])
    alpha = _relu(jnp.dot(v, A2T[...], preferred_element_type=jnp.float32)
                  + ab2[...])
    ex = jnp.exp(alpha)
    msg = ex * (xl + delta)
    out_ref[0] = jnp.concatenate([ex[:, :64], msg[:, :64]], axis=1)
    out_ref[1] = jnp.concatenate([ex[:, 64:], msg[:, 64:]], axis=1)


def _emlp_call(eD, eS, pb1, P2T, pb2, A1T, ab1, A2T, ab2):
    nb = EC // EPB
    full = lambda a: pl.BlockSpec(a.shape, lambda i: (0,) * a.ndim)
    return pl.pallas_call(
        _emlp_body,
        grid=(nb,),
        in_specs=[pl.BlockSpec((EPB, D), lambda i: (i, 0)),
                  pl.BlockSpec((EPB, D), lambda i: (i, 0))] + [
            full(a) for a in (pb1, P2T, pb2, A1T, ab1, A2T, ab2)],
        out_specs=pl.BlockSpec((2, EPB, D), lambda i: (0, i, 0)),
        out_shape=jax.ShapeDtypeStruct((2, EC, D), jnp.float32),
    )(eD, eS, pb1, P2T, pb2, A1T, ab1, A2T, ab2)


_RSLICE = 624


_EPS = EC // NSUB
_SU = _EPS // UNIT
_ST = _EPS - _SU * UNIT


@functools.partial(
    pl.kernel,
    mesh=_sc_mesh,
    out_type=jax.ShapeDtypeStruct((2, N, D), jnp.float32),
    scratch_types=[pltpu.VMEM((UNIT,), jnp.int32),
                   pltpu.VMEM((UNIT,), jnp.int32),
                   pltpu.VMEM((_EPS - (_EPS // UNIT) * UNIT,), jnp.int32),
                   pltpu.VMEM((UNIT, D), jnp.float32),
                   pltpu.VMEM((UNIT, D), jnp.float32),
                   pltpu.VMEM_SHARED((N, D), jnp.float32),
                   pltpu.SemaphoreType.DMA,
                   pltpu.SemaphoreType.DMA],
)
def _sc_scatter(dst1, em3, zeros, accO, id0, id1, idt, bE0, bE1, acc_sh,
                sem0, sem1):
    c = lax.axis_index("c")
    s = lax.axis_index("s")
    r0 = s * _RSLICE
    pltpu.sync_copy(zeros, acc_sh.at[pl.ds(r0, _RSLICE), :])

    @pl.when(s == NSUB - 1)
    def _():
        pltpu.sync_copy(zeros.at[pl.ds(0, 16), :],
                        acc_sh.at[pl.ds(NSUB * _RSLICE, 16), :])

    plsc.subcore_barrier()
    base = pl.multiple_of(s * _EPS, 8)
    off_u = lambda u: pl.multiple_of(base + u * UNIT, 8)

    def start(off, idb, bE, sem):
        pltpu.sync_copy(dst1.at[pl.ds(off, UNIT)], idb)
        pltpu.async_copy(em3.at[c, pl.ds(off, UNIT), :], bE, sem)

    def finish(off, idb, bE, sem):
        pltpu.make_async_copy(em3.at[c, pl.ds(off, UNIT), :], bE,
                              sem).wait()
        pltpu.sync_copy(bE, acc_sh.at[idb], add=True)

    start(off_u(0), id0, bE0, sem0)

    def body(t, carry):
        u = 2 * t
        start(off_u(u + 1), id1, bE1, sem1)
        finish(off_u(u), id0, bE0, sem0)
        start(off_u(u + 2), id0, bE0, sem0)
        finish(off_u(u + 1), id1, bE1, sem1)
        return carry

    lax.fori_loop(0, _SU // 2 - 1, body, 0)
    u = _SU - 2
    start(off_u(u + 1), id1, bE1, sem1)
    finish(off_u(u), id0, bE0, sem0)
    finish(off_u(u + 1), id1, bE1, sem1)
    toff = pl.multiple_of(base + _SU * UNIT, 8)
    pltpu.sync_copy(dst1.at[pl.ds(toff, _ST)], idt)
    pltpu.sync_copy(em3.at[c, pl.ds(toff, _ST), :], bE0.at[pl.ds(0, _ST), :])
    pltpu.sync_copy(bE0.at[pl.ds(0, _ST), :], acc_sh.at[idt], add=True)
    plsc.subcore_barrier()
    pltpu.sync_copy(acc_sh.at[pl.ds(r0, _RSLICE), :],
                    accO.at[c, pl.ds(r0, _RSLICE), :])

    @pl.when(s == NSUB - 1)
    def _():
        pltpu.sync_copy(acc_sh.at[pl.ds(NSUB * _RSLICE, 16), :],
                        accO.at[c, pl.ds(NSUB * _RSLICE, 16), :])


def _final_body(*refs):
    acc_refs = refs[:K]
    den0_ref, num0_ref, WoutT, b_out, o_ref = refs[K:]
    acc = acc_refs[0][...]
    for r in acc_refs[1:]:
        acc = acc + r[...]
    den = jnp.concatenate([acc[0, :, :64], acc[1, :, :64]], axis=1)
    num = jnp.concatenate([acc[0, :, 64:], acc[1, :, 64:]], axis=1)
    den = den + den0_ref[...] + 1e-16
    num = num + num0_ref[...]
    o_ref[...] = _relu(jnp.dot(num / den, WoutT[...],
                               preferred_element_type=jnp.float32)
                       + b_out[...])


def _final_call(accs, den0, num0, WoutT, b_out):
    nb = N // NPB
    full = lambda a: pl.BlockSpec(a.shape, lambda i: (0,) * a.ndim)
    row_spec = pl.BlockSpec((NPB, D), lambda i: (i, 0))
    acc_spec = pl.BlockSpec((2, NPB, D), lambda i: (0, i, 0))
    return pl.pallas_call(
        _final_body,
        grid=(nb,),
        in_specs=[acc_spec] * K + [row_spec, row_spec, full(WoutT),
                                   full(b_out)],
        out_specs=row_spec,
        out_shape=jax.ShapeDtypeStruct((N, D), jnp.float32),
    )(*accs, den0, num0, WoutT, b_out)


def kernel(x, pos, edge_index, Win, b_in, Wout, b_out, Wlin, Wsrc, Wdst,
           P1, pb1, P2, pb2, A1, ab1, A2, ab2):
    posp = jnp.pad(pos, ((0, 0), (0, 5)))
    P1pT = jnp.pad(P1, ((0, 0), (0, 5))).T
    row = lambda v: v.reshape(1, -1)
    T1, T2, den0, num0 = _prep_call(
        x, posp, Win.T, row(b_in), Wlin.T, Wsrc.T, Wdst.T, P1pT, row(pb1),
        P2.T, row(pb2), A1.T, row(ab1), A2.T, row(ab2))
    src1 = edge_index[0]
    dst1 = edge_index[1]
    zeros = jnp.zeros((_RSLICE, D), jnp.float32)
    accs = []
    for k in range(K):
        sl = slice(k * EC, (k + 1) * EC)
        eD, eS = _sc_gather(src1[sl], dst1[sl], T1, T2)
        em3 = _emlp_call(eD, eS, row(pb1), P2.T, row(pb2), A1.T, row(ab1),
                         A2.T, row(ab2))
        accs.append(_sc_scatter(dst1[sl], em3, zeros))
    return _final_call(accs, den0, num0, Wout.T, row(b_out))


# speedup vs baseline: 1.7470x; 1.0002x over previous
import functools

import jax
import jax.numpy as jnp
from jax import lax
from jax.experimental import pallas as pl
from jax.experimental.pallas import tpu as pltpu
from jax.experimental.pallas import tpu_sc as plsc

N = 10000
E = 320000
D = 128
UNIT = 128
R = E // UNIT
NSC = 2
NSUB = 16
NW = NSC * NSUB
NPB = 400
EPB = 1600
K = 2
EC = E // K
RC = EC // UNIT

_relu = jax.nn.relu


def _pack(a):
    n = a.shape[1] // 2
    b = a.astype(jnp.bfloat16)
    u = lax.convert_element_type(
        lax.bitcast_convert_type(b, jnp.uint16), jnp.uint32)
    w = u[:, :n] | (u[:, n:] << 16)
    return lax.bitcast_convert_type(w, jnp.float32)


def _unpack(p):
    u = lax.bitcast_convert_type(p, jnp.uint32)
    lo = lax.bitcast_convert_type(u << 16, jnp.float32)
    hi = lax.bitcast_convert_type(u & jnp.uint32(0xFFFF0000), jnp.float32)
    return jnp.concatenate([lo, hi], axis=1)


def _prep_body(x_ref, posp_ref, WinT, b_in, WlinT, WsrcT, WdstT, P1pT, pb1,
               P2T, pb2, A1T, ab1, A2T, ab2,
               T1_ref, T2_ref, den0_ref, num0_ref):
    x = x_ref[...]
    h = _relu(jnp.dot(x, WinT[...], preferred_element_type=jnp.float32)
              + b_in[...])
    xl = jnp.dot(h, WlinT[...], preferred_element_type=jnp.float32)
    dd = jnp.dot(jnp.dot(h, WdstT[...], preferred_element_type=jnp.float32),
                 A1T[...], preferred_element_type=jnp.float32)
    ss = jnp.dot(jnp.dot(h, WsrcT[...], preferred_element_type=jnp.float32),
                 A1T[...], preferred_element_type=jnp.float32)
    q = jnp.dot(posp_ref[...], P1pT[...], preferred_element_type=jnp.float32)
    dl64 = _relu(pb1[...])
    dl128 = _relu(jnp.dot(dl64, P2T[...],
                          preferred_element_type=jnp.float32) + pb2[...])
    v0 = _relu(jnp.dot(dl128, A1T[...], preferred_element_type=jnp.float32)
               + dd - ss + ab1[...])
    alpha0 = _relu(jnp.dot(v0, A2T[...], preferred_element_type=jnp.float32)
                   + ab2[...])
    ex0 = jnp.exp(alpha0)
    den0_ref[...] = ex0
    num0_ref[...] = ex0 * (xl + dl128)
    T1_ref[...] = jnp.concatenate([q, dd], axis=1)
    T2_ref[...] = jnp.concatenate([_pack(xl), _pack(q), _pack(ss)], axis=1)


def _prep_call(x, posp, WinT, b_in, WlinT, WsrcT, WdstT, P1pT, pb1, P2T, pb2,
               A1T, ab1, A2T, ab2):
    nb = N // NPB
    full = lambda a: pl.BlockSpec(a.shape, lambda i: (0,) * a.ndim)
    row_spec = lambda w: pl.BlockSpec((NPB, w), lambda i: (i, 0))
    return pl.pallas_call(
        _prep_body,
        grid=(nb,),
        in_specs=[row_spec(D), row_spec(8)] + [
            full(a) for a in (WinT, b_in, WlinT, WsrcT, WdstT, P1pT, pb1,
                              P2T, pb2, A1T, ab1, A2T, ab2)],
        out_specs=[row_spec(D), row_spec(D), row_spec(D), row_spec(D)],
        out_shape=[jax.ShapeDtypeStruct((N, D), jnp.float32),
                   jax.ShapeDtypeStruct((N, D), jnp.float32),
                   jax.ShapeDtypeStruct((N, D), jnp.float32),
                   jax.ShapeDtypeStruct((N, D), jnp.float32)],
    )(x, posp, WinT, b_in, WlinT, WsrcT, WdstT, P1pT, pb1, P2T, pb2,
      A1T, ab1, A2T, ab2)


_sc_mesh = plsc.VectorSubcoreMesh(core_axis_name="c", subcore_axis_name="s")


_EPW = EC // NW
_GU = _EPW // UNIT
_GT = _EPW - _GU * UNIT


@functools.partial(
    pl.kernel,
    mesh=_sc_mesh,
    out_type=[jax.ShapeDtypeStruct((EC, D), jnp.float32),
              jax.ShapeDtypeStruct((EC, D), jnp.float32)],
    scratch_types=[pltpu.VMEM((UNIT,), jnp.int32),
                   pltpu.VMEM((UNIT,), jnp.int32),
                   pltpu.VMEM((UNIT,), jnp.int32),
                   pltpu.VMEM((UNIT,), jnp.int32),
                   pltpu.VMEM((UNIT, D), jnp.float32),
                   pltpu.VMEM((UNIT, D), jnp.float32),
                   pltpu.VMEM((UNIT, D), jnp.float32),
                   pltpu.VMEM((UNIT, D), jnp.float32),
                   pltpu.SemaphoreType.DMA,
                   pltpu.SemaphoreType.DMA,
                   pltpu.SemaphoreType.DMA,
                   pltpu.SemaphoreType.DMA],
)
def _sc_gather(src1, dst1, T1, T2, outD, outS, is0, id0, is1, id1,
               bD0, bS0, bD1, bS1, sem0, sem1, so0, so1):
    c = lax.axis_index("c")
    s = lax.axis_index("s")
    wid = s * NSC + c
    base = pl.multiple_of(wid * _EPW, 8)

    def start(off, isb, idb, bD, bS, sem):
        pltpu.sync_copy(src1.at[pl.ds(off, UNIT)], isb)
        pltpu.sync_copy(dst1.at[pl.ds(off, UNIT)], idb)
        pltpu.async_copy(T1.at[idb], bD, sem)
        pltpu.async_copy(T2.at[isb], bS, sem)

    def drain_stores(off, bD, bS, so):
        pltpu.make_async_copy(bD, outD.at[pl.ds(off, UNIT), :], so).wait()
        pltpu.make_async_copy(bS, outS.at[pl.ds(off, UNIT), :], so).wait()

    def start2(off, prev_off, isb, idb, bD, bS, sem, so):
        drain_stores(prev_off, bD, bS, so)
        start(off, isb, idb, bD, bS, sem)

    def finish(off, isb, idb, bD, bS, sem, so):
        pltpu.make_async_copy(T1.at[idb], bD, sem).wait()
        pltpu.make_async_copy(T2.at[isb], bS, sem).wait()
        pltpu.async_copy(bD, outD.at[pl.ds(off, UNIT), :], so)
        pltpu.async_copy(bS, outS.at[pl.ds(off, UNIT), :], so)

    off_u = lambda u: pl.multiple_of(base + u * UNIT, 8)
    start(off_u(0), is0, id0, bD0, bS0, sem0)
    start(off_u(1), is1, id1, bD1, bS1, sem1)

    def body(t, carry):
        u = 2 * t
        finish(off_u(u), is0, id0, bD0, bS0, sem0, so0)
        start2(off_u(u + 2), off_u(u), is0, id0, bD0, bS0, sem0, so0)
        finish(off_u(u + 1), is1, id1, bD1, bS1, sem1, so1)
        start2(off_u(u + 3), off_u(u + 1), is1, id1, bD1, bS1, sem1, so1)
        return carry

    lax.fori_loop(0, _GU // 2 - 1, body, 0)
    finish(off_u(_GU - 3), is0, id0, bD0, bS0, sem0, so0)
    start2(off_u(_GU - 1), off_u(_GU - 3), is0, id0, bD0, bS0, sem0, so0)
    finish(off_u(_GU - 2), is1, id1, bD1, bS1, sem1, so1)
    finish(off_u(_GU - 1), is0, id0, bD0, bS0, sem0, so0)
    drain_stores(off_u(_GU - 2), bD1, bS1, so1)
    drain_stores(off_u(_GU - 1), bD0, bS0, so0)
    toff = pl.multiple_of(base + _GU * UNIT, 8)
    pltpu.sync_copy(src1.at[pl.ds(toff, _GT)], is1.at[pl.ds(0, _GT)])
    pltpu.sync_copy(dst1.at[pl.ds(toff, _GT)], id1.at[pl.ds(0, _GT)])
    cp1 = pltpu.async_copy(T1.at[id1.at[pl.ds(0, _GT)]],
                           bD1.at[pl.ds(0, _GT), :], sem1)
    cp2 = pltpu.async_copy(T2.at[is1.at[pl.ds(0, _GT)]],
                           bS1.at[pl.ds(0, _GT), :], sem1)
    cp1.wait()
    cp2.wait()
    pltpu.sync_copy(bD1.at[pl.ds(0, _GT), :], outD.at[pl.ds(toff, _GT), :])
    pltpu.sync_copy(bS1.at[pl.ds(0, _GT), :], outS.at[pl.ds(toff, _GT), :])


def _emlp_body(eD_ref, eS_ref, pb1, P2T, pb2, A1T, ab1, A2T, ab2, out_ref):
    eD = eD_ref[...]
    eS = eS_ref[...]
    qd = eD[:, :64]
    dd = eD[:, 64:]
    xl = _unpack(eS[:, :64])
    qs = _unpack(eS[:, 64:96])
    ss = _unpack(eS[:, 96:128])
    e1 = _relu(qd - qs + pb1[...])
    delta = _relu(jnp.dot(e1, P2T[...], preferred_element_type=jnp.float32)
                  + pb2[...])
    v = _relu(jnp.dot(delta, A1T[...], preferred_element_type=jnp.float32)
              + (dd - ss) + ab1[...])
    alpha = _relu(jnp.dot(v, A2T[...], preferred_element_type=jnp.float32)
                  + ab2[...])
    ex = jnp.exp(alpha)
    msg = ex * (xl + delta)
    out_ref[0] = jnp.concatenate([ex[:, :64], msg[:, :64]], axis=1)
    out_ref[1] = jnp.concatenate([ex[:, 64:], msg[:, 64:]], axis=1)


def _emlp_call(eD, eS, pb1, P2T, pb2, A1T, ab1, A2T, ab2):
    nb = EC // EPB
    full = lambda a: pl.BlockSpec(a.shape, lambda i: (0,) * a.ndim)
    return pl.pallas_call(
        _emlp_body,
        grid=(nb,),
        in_specs=[pl.BlockSpec((EPB, D), lambda i: (i, 0)),
                  pl.BlockSpec((EPB, D), lambda i: (i, 0))] + [
            full(a) for a in (pb1, P2T, pb2, A1T, ab1, A2T, ab2)],
        out_specs=pl.BlockSpec((2, EPB, D), lambda i: (0, i, 0)),
        out_shape=jax.ShapeDtypeStruct((2, EC, D), jnp.float32),
    )(eD, eS, pb1, P2T, pb2, A1T, ab1, A2T, ab2)


_RSLICE = 624


_EPS = EC // NSUB
_SU = _EPS // UNIT
_ST = _EPS - _SU * UNIT


@functools.partial(
    pl.kernel,
    mesh=_sc_mesh,
    out_type=jax.ShapeDtypeStruct((2, N, D), jnp.float32),
    scratch_types=[pltpu.VMEM((UNIT,), jnp.int32),
                   pltpu.VMEM((UNIT,), jnp.int32),
                   pltpu.VMEM((_EPS - (_EPS // UNIT) * UNIT,), jnp.int32),
                   pltpu.VMEM((UNIT, D), jnp.float32),
                   pltpu.VMEM((UNIT, D), jnp.float32),
                   pltpu.VMEM_SHARED((N, D), jnp.float32),
                   pltpu.SemaphoreType.DMA,
                   pltpu.SemaphoreType.DMA,
                   pltpu.SemaphoreType.DMA,
                   pltpu.SemaphoreType.DMA],
)
def _sc_scatter(dst1, em3, zeros, accO, id0, id1, idt, bE0, bE1, acc_sh,
                sem0, sem1, sa0, sa1):
    c = lax.axis_index("c")
    s = lax.axis_index("s")
    r0 = s * _RSLICE
    pltpu.sync_copy(zeros, acc_sh.at[pl.ds(r0, _RSLICE), :])

    @pl.when(s == NSUB - 1)
    def _():
        pltpu.sync_copy(zeros.at[pl.ds(0, 16), :],
                        acc_sh.at[pl.ds(NSUB * _RSLICE, 16), :])

    plsc.subcore_barrier()
    base = pl.multiple_of(s * _EPS, 8)
    off_u = lambda u: pl.multiple_of(base + u * UNIT, 8)

    def start(off, idb, bE, sem):
        pltpu.sync_copy(dst1.at[pl.ds(off, UNIT)], idb)
        pltpu.async_copy(em3.at[c, pl.ds(off, UNIT), :], bE, sem)

    def drain_add(idb, bE, sa):
        pltpu.make_async_copy(bE, acc_sh.at[idb], sa).wait()

    def start2(off, idb, bE, sem, sa):
        drain_add(idb, bE, sa)
        start(off, idb, bE, sem)

    def finish(off, idb, bE, sem, sa):
        pltpu.make_async_copy(em3.at[c, pl.ds(off, UNIT), :], bE,
                              sem).wait()
        pltpu.async_copy(bE, acc_sh.at[idb], sa, add=True)

    start(off_u(0), id0, bE0, sem0)
    start(off_u(1), id1, bE1, sem1)

    def body(t, carry):
        u = 2 * t
        finish(off_u(u), id0, bE0, sem0, sa0)
        start2(off_u(u + 2), id0, bE0, sem0, sa0)
        finish(off_u(u + 1), id1, bE1, sem1, sa1)
        start2(off_u(u + 3), id1, bE1, sem1, sa1)
        return carry

    lax.fori_loop(0, _SU // 2 - 2, body, 0)
    finish(off_u(_SU - 4), id0, bE0, sem0, sa0)
    start2(off_u(_SU - 2), id0, bE0, sem0, sa0)
    finish(off_u(_SU - 3), id1, bE1, sem1, sa1)
    start2(off_u(_SU - 1), id1, bE1, sem1, sa1)
    finish(off_u(_SU - 2), id0, bE0, sem0, sa0)
    finish(off_u(_SU - 1), id1, bE1, sem1, sa1)
    drain_add(id0, bE0, sa0)
    drain_add(id1, bE1, sa1)
    toff = pl.multiple_of(base + _SU * UNIT, 8)
    pltpu.sync_copy(dst1.at[pl.ds(toff, _ST)], idt)
    pltpu.sync_copy(em3.at[c, pl.ds(toff, _ST), :], bE0.at[pl.ds(0, _ST), :])
    pltpu.sync_copy(bE0.at[pl.ds(0, _ST), :], acc_sh.at[idt], add=True)
    plsc.subcore_barrier()
    pltpu.sync_copy(acc_sh.at[pl.ds(r0, _RSLICE), :],
                    accO.at[c, pl.ds(r0, _RSLICE), :])

    @pl.when(s == NSUB - 1)
    def _():
        pltpu.sync_copy(acc_sh.at[pl.ds(NSUB * _RSLICE, 16), :],
                        accO.at[c, pl.ds(NSUB * _RSLICE, 16), :])


def _final_body(*refs):
    acc_refs = refs[:K]
    den0_ref, num0_ref, WoutT, b_out, o_ref = refs[K:]
    acc = acc_refs[0][...]
    for r in acc_refs[1:]:
        acc = acc + r[...]
    den = jnp.concatenate([acc[0, :, :64], acc[1, :, :64]], axis=1)
    num = jnp.concatenate([acc[0, :, 64:], acc[1, :, 64:]], axis=1)
    den = den + den0_ref[...] + 1e-16
    num = num + num0_ref[...]
    o_ref[...] = _relu(jnp.dot(num / den, WoutT[...],
                               preferred_element_type=jnp.float32)
                       + b_out[...])


def _final_call(accs, den0, num0, WoutT, b_out):
    nb = N // NPB
    full = lambda a: pl.BlockSpec(a.shape, lambda i: (0,) * a.ndim)
    row_spec = pl.BlockSpec((NPB, D), lambda i: (i, 0))
    acc_spec = pl.BlockSpec((2, NPB, D), lambda i: (0, i, 0))
    return pl.pallas_call(
        _final_body,
        grid=(nb,),
        in_specs=[acc_spec] * K + [row_spec, row_spec, full(WoutT),
                                   full(b_out)],
        out_specs=row_spec,
        out_shape=jax.ShapeDtypeStruct((N, D), jnp.float32),
    )(*accs, den0, num0, WoutT, b_out)


def kernel(x, pos, edge_index, Win, b_in, Wout, b_out, Wlin, Wsrc, Wdst,
           P1, pb1, P2, pb2, A1, ab1, A2, ab2):
    posp = jnp.pad(pos, ((0, 0), (0, 5)))
    P1pT = jnp.pad(P1, ((0, 0), (0, 5))).T
    row = lambda v: v.reshape(1, -1)
    T1, T2, den0, num0 = _prep_call(
        x, posp, Win.T, row(b_in), Wlin.T, Wsrc.T, Wdst.T, P1pT, row(pb1),
        P2.T, row(pb2), A1.T, row(ab1), A2.T, row(ab2))
    src1 = edge_index[0]
    dst1 = edge_index[1]
    zeros = jnp.zeros((_RSLICE, D), jnp.float32)
    accs = []
    for k in range(K):
        sl = slice(k * EC, (k + 1) * EC)
        eD, eS = _sc_gather(src1[sl], dst1[sl], T1, T2)
        em3 = _emlp_call(eD, eS, row(pb1), P2.T, row(pb2), A1.T, row(ab1),
                         A2.T, row(ab2))
        accs.append(_sc_scatter(dst1[sl], em3, zeros))
    return _final_call(accs, den0, num0, Wout.T, row(b_out))
